# baseline (speedup 1.0000x reference)
import functools
import math

import jax
import jax.numpy as jnp
from jax import lax
from jax.experimental import pallas as pl
from jax.experimental.pallas import tpu as pltpu
from jax.experimental.pallas import tpu_sc as plsc

N_HID = 128
MAX_LEN = 100
E = 320000

_NC = 2
_NS = 16
_NW = _NC * _NS

_CHUNK = 80


def _fold_body(emb_ref, w_ref, b_ref, out_ref):
    out_ref[...] = lax.dot_general(
        emb_ref[...], w_ref[...],
        dimension_numbers=(((1,), (1,)), ((), ())),
        preferred_element_type=jnp.float32,
    ) + b_ref[...]


def _fold_table(emb, W, b):
    return pl.pallas_call(
        _fold_body,
        out_shape=jax.ShapeDtypeStruct(emb.shape, jnp.float32),
    )(emb, W, b.reshape(1, N_HID))


_NSLOT = 4

_PER_W = 69
_E_SC = _NW * _CHUNK * _PER_W
_R_TC = 2048
_E_TC = E - _E_SC
_ROWS_W = _E_SC // _NW


def _sc_gather_body(table_hbm, t_hbm, out_hbm, tab_sh, idx_v, buf, gsem, ssem):
    sid = lax.axis_index("s")
    wid = sid * _NC + lax.axis_index("c")
    out_base = wid * _ROWS_W

    @pl.when(sid == 0)
    def _():
        pltpu.sync_copy(table_hbm, tab_sh)

    pltpu.sync_copy(t_hbm.at[pl.ds(out_base, _ROWS_W)], idx_v)
    plsc.subcore_barrier()

    def start_gather(j, slot):
        idx_chunk = idx_v.at[pl.ds(j * _CHUNK, _CHUNK)]
        pltpu.async_copy(tab_sh.at[idx_chunk], buf.at[slot], gsem.at[slot])

    def start_store(j, slot):
        pltpu.async_copy(
            buf.at[slot], out_hbm.at[pl.ds(out_base + j * _CHUNK, _CHUNK)],
            ssem.at[slot])

    def wait_gather(slot):
        pltpu.make_async_copy(
            tab_sh.at[idx_v.at[pl.ds(0, _CHUNK)]], buf.at[slot],
            gsem.at[slot]).wait()

    def wait_store(slot):
        pltpu.make_async_copy(
            buf.at[slot], out_hbm.at[pl.ds(out_base, _CHUNK)],
            ssem.at[slot]).wait()

    for k in range(_NSLOT - 1):
        start_gather(k, k)

    def body(j, carry):
        slot = lax.rem(j, _NSLOT)

        @pl.when(j > 0)
        def _():
            wait_store(lax.rem(j - 1, _NSLOT))

        @pl.when(j + _NSLOT - 1 < _PER_W)
        def _():
            start_gather(j + _NSLOT - 1, lax.rem(j + _NSLOT - 1, _NSLOT))

        wait_gather(slot)
        start_store(j, slot)
        return carry

    lax.fori_loop(0, _PER_W, body, 0)
    wait_store((_PER_W - 1) % _NSLOT)


def _tc_gather_body(t_ref, tab_ref, out_ref):
    tb = t_ref[0, 0, :]
    cols = lax.broadcasted_iota(jnp.int32, (_R_TC, N_HID), 1)
    oh = (lax.broadcast_in_dim(tb, (_R_TC, N_HID), (0,)) == cols)
    out_ref[...] = lax.dot_general(
        oh.astype(jnp.float32), tab_ref[...],
        dimension_numbers=(((1,), (0,)), ((), ())),
        preferred_element_type=jnp.float32,
    )


def _tc_gather(t_tc, table_pad):
    t3d = t_tc.reshape(_E_TC // _R_TC, 1, _R_TC)
    return pl.pallas_call(
        _tc_gather_body,
        grid=(_E_TC // _R_TC,),
        in_specs=[
            pl.BlockSpec((1, 1, _R_TC), lambda i: (i, 0, 0)),
            pl.BlockSpec((N_HID, N_HID), lambda i: (0, 0)),
        ],
        out_specs=pl.BlockSpec((_R_TC, N_HID), lambda i: (i, 0)),
        out_shape=jax.ShapeDtypeStruct((_E_TC, N_HID), jnp.float32),
    )(t3d, table_pad)


def kernel(x, t, emb, W, b):
    del x
    table2 = _fold_table(emb, W, b)
    t32 = t.astype(jnp.int32)

    mesh = plsc.VectorSubcoreMesh(
        core_axis_name="c", subcore_axis_name="s",
        num_cores=_NC, num_subcores=_NS,
    )
    gather = functools.partial(
        pl.kernel,
        out_type=jax.ShapeDtypeStruct((_E_SC, N_HID), jnp.float32),
        mesh=mesh,
        scratch_types=[
            pltpu.VMEM_SHARED((MAX_LEN, N_HID), jnp.float32),
            pltpu.VMEM((_ROWS_W,), jnp.int32),
            pltpu.VMEM((_NSLOT, _CHUNK, N_HID), jnp.float32),
            pltpu.SemaphoreType.DMA((_NSLOT,)),
            pltpu.SemaphoreType.DMA((_NSLOT,)),
        ],
    )(_sc_gather_body)
    sc_out = gather(table2, t32[:_E_SC])

    table_pad = jnp.pad(table2, ((0, N_HID - MAX_LEN), (0, 0)))
    tc_out = _tc_gather(t32[_E_SC:], table_pad)
    return jnp.concatenate([sc_out, tc_out], axis=0)


# speedup vs baseline: 2.3530x; 2.3530x over previous
import functools
import math

import jax
import jax.numpy as jnp
from jax import lax
from jax.experimental import pallas as pl
from jax.experimental.pallas import tpu as pltpu
from jax.experimental.pallas import tpu_sc as plsc

N_HID = 128
MAX_LEN = 100
E = 320000

_NC = 2
_NS = 16
_NW = _NC * _NS

_CHUNK = 80
_NROWS = E // _CHUNK
_PER_W = _NROWS // _NW


def _fold_body(emb_ref, w_ref, b_ref, out_ref):
    out_ref[...] = lax.dot_general(
        emb_ref[...], w_ref[...],
        dimension_numbers=(((1,), (1,)), ((), ())),
        preferred_element_type=jnp.float32,
    ) + b_ref[...]


def _fold_table(emb, W, b):
    return pl.pallas_call(
        _fold_body,
        out_shape=jax.ShapeDtypeStruct(emb.shape, jnp.float32),
    )(emb, W, b.reshape(1, N_HID))


_ROWS_W = E // _NW
_NSLOT = 8


def _sc_gather_body(table_hbm, t_hbm, out_hbm, tab_sh, idx_v, buf, gsem, ssem):
    sid = lax.axis_index("s")
    wid = sid * _NC + lax.axis_index("c")
    out_base = wid * _ROWS_W

    @pl.when(sid == 0)
    def _():
        pltpu.sync_copy(table_hbm, tab_sh)

    pltpu.sync_copy(t_hbm.at[pl.ds(out_base, _ROWS_W)], idx_v)
    plsc.subcore_barrier()

    def start_gather(j, slot):
        idx_chunk = idx_v.at[pl.ds(j * _CHUNK, _CHUNK)]
        pltpu.async_copy(tab_sh.at[idx_chunk], buf.at[slot], gsem.at[slot])

    def start_store(j, slot):
        pltpu.async_copy(
            buf.at[slot], out_hbm.at[pl.ds(out_base + j * _CHUNK, _CHUNK)],
            ssem.at[slot])

    def wait_gather(slot):
        pltpu.make_async_copy(
            tab_sh.at[idx_v.at[pl.ds(0, _CHUNK)]], buf.at[slot],
            gsem.at[slot]).wait()

    def wait_store(slot):
        pltpu.make_async_copy(
            buf.at[slot], out_hbm.at[pl.ds(out_base, _CHUNK)],
            ssem.at[slot]).wait()

    for k in range(_NSLOT - 1):
        start_gather(k, k)

    def body(j, carry):
        slot = lax.rem(j, _NSLOT)

        @pl.when(j > 0)
        def _():
            wait_store(lax.rem(j - 1, _NSLOT))

        @pl.when(j + _NSLOT - 1 < _PER_W)
        def _():
            start_gather(j + _NSLOT - 1, lax.rem(j + _NSLOT - 1, _NSLOT))

        wait_gather(slot)
        start_store(j, slot)
        return carry

    lax.fori_loop(0, _PER_W, body, 0)
    wait_store((_PER_W - 1) % _NSLOT)


def kernel(x, t, emb, W, b):
    del x
    table2 = _fold_table(emb, W, b)
    t32 = t.astype(jnp.int32)

    mesh = plsc.VectorSubcoreMesh(
        core_axis_name="c", subcore_axis_name="s",
        num_cores=_NC, num_subcores=_NS,
    )
    gather = functools.partial(
        pl.kernel,
        out_type=jax.ShapeDtypeStruct((E, N_HID), jnp.float32),
        mesh=mesh,
        scratch_types=[
            pltpu.VMEM_SHARED((MAX_LEN, N_HID), jnp.float32),
            pltpu.VMEM((_ROWS_W,), jnp.int32),
            pltpu.VMEM((_NSLOT, _CHUNK, N_HID), jnp.float32),
            pltpu.SemaphoreType.DMA((_NSLOT,)),
            pltpu.SemaphoreType.DMA((_NSLOT,)),
        ],
    )(_sc_gather_body)
    return gather(table2, t32)


# speedup vs baseline: 2.3635x; 1.0045x over previous
import functools

import jax
import jax.numpy as jnp
from jax import lax
from jax.experimental import pallas as pl
from jax.experimental.pallas import tpu as pltpu
from jax.experimental.pallas import tpu_sc as plsc

N_HID = 128
MAX_LEN = 100
E = 320000

_NC = 2
_NS = 16
_NW = _NC * _NS

_CHUNK = 80
_NROWS = E // _CHUNK
_PER_W = _NROWS // _NW


def _fold_body(emb_ref, w_ref, b_ref, out_ref):
    out_ref[...] = lax.dot_general(
        emb_ref[...], w_ref[...],
        dimension_numbers=(((1,), (1,)), ((), ())),
        preferred_element_type=jnp.float32,
    ) + b_ref[...]


def _fold_table(emb, W, b):
    return pl.pallas_call(
        _fold_body,
        out_shape=jax.ShapeDtypeStruct(emb.shape, jnp.float32),
    )(emb, W, b.reshape(1, N_HID))


_ROWS_W = E // _NW
_NSLOT = 4


def _sc_gather_body(table_hbm, t_hbm, out_hbm, tab_sh, idx_v, buf, gsem, ssem):
    sid = lax.axis_index("s")
    wid = sid * _NC + lax.axis_index("c")
    out_base = wid * _ROWS_W

    @pl.when(sid == 0)
    def _():
        pltpu.sync_copy(table_hbm, tab_sh)

    pltpu.sync_copy(t_hbm.at[pl.ds(out_base, _ROWS_W)], idx_v)
    plsc.subcore_barrier()

    def start_gather(j, slot):
        idx_chunk = idx_v.at[pl.ds(j * _CHUNK, _CHUNK)]
        pltpu.async_copy(tab_sh.at[idx_chunk], buf.at[slot], gsem.at[slot])

    def start_store(j, slot):
        pltpu.async_copy(
            buf.at[slot], out_hbm.at[pl.ds(out_base + j * _CHUNK, _CHUNK)],
            ssem.at[slot])

    def wait_gather(slot):
        pltpu.make_async_copy(
            tab_sh.at[idx_v.at[pl.ds(0, _CHUNK)]], buf.at[slot],
            gsem.at[slot]).wait()

    def wait_store(slot):
        pltpu.make_async_copy(
            buf.at[slot], out_hbm.at[pl.ds(out_base, _CHUNK)],
            ssem.at[slot]).wait()

    for k in range(_NSLOT - 1):
        start_gather(k, k)

    def body(j, carry):
        slot = lax.rem(j, _NSLOT)

        @pl.when(j > 0)
        def _():
            wait_store(lax.rem(j - 1, _NSLOT))

        @pl.when(j + _NSLOT - 1 < _PER_W)
        def _():
            start_gather(j + _NSLOT - 1, lax.rem(j + _NSLOT - 1, _NSLOT))

        wait_gather(slot)
        start_store(j, slot)
        return carry

    lax.fori_loop(0, _PER_W, body, 0)
    wait_store((_PER_W - 1) % _NSLOT)


def kernel(x, t, emb, W, b):
    del x
    table2 = _fold_table(emb, W, b)
    t32 = t.astype(jnp.int32)

    mesh = plsc.VectorSubcoreMesh(
        core_axis_name="c", subcore_axis_name="s",
        num_cores=_NC, num_subcores=_NS,
    )
    gather = functools.partial(
        pl.kernel,
        out_type=jax.ShapeDtypeStruct((E, N_HID), jnp.float32),
        mesh=mesh,
        scratch_types=[
            pltpu.VMEM_SHARED((MAX_LEN, N_HID), jnp.float32),
            pltpu.VMEM((_ROWS_W,), jnp.int32),
            pltpu.VMEM((_NSLOT, _CHUNK, N_HID), jnp.float32),
            pltpu.SemaphoreType.DMA((_NSLOT,)),
            pltpu.SemaphoreType.DMA((_NSLOT,)),
        ],
    )(_sc_gather_body)
    return gather(table2, t32)
